# Initial kernel scaffold; baseline (speedup 1.0000x reference)
import jax
import jax.numpy as jnp
from jax.experimental import pallas as pl


def kernel(x, edge_index, edge_weight, W1, b1, gamma, beta, running_mean, running_var, W2, b2):
    raise NotImplementedError("write your pallas kernel here")


# speedup vs baseline: 2.3962x; 2.3962x over previous
import dataclasses
import functools

import jax
import jax.numpy as jnp
from jax import lax
from jax.experimental import pallas as pl
from jax.experimental.pallas import tpu as pltpu
from jax.experimental.pallas import tpu_sc as plsc

N = 50000
E = 800000
D_IN = 128
D_HID = 128
D_OUT = 64
HALF = D_OUT // 2
K = 10
ALPHA = 0.5

NUM_CORES = 2
NUM_SUBCORES = 16
L = 16

CHUNK = 128
CHUNKS_PER_TILE = 400
E_PAD = CHUNKS_PER_TILE * NUM_SUBCORES * CHUNK
N_PAD = 50048
ROWS_PER_TILE = N_PAD // NUM_SUBCORES

ROW_BLOCK = 1000


def _mlp_body(x_ref, w1_ref, b1_ref, g_ref, bb_ref, m_ref, v_ref, w2_ref,
              b2_ref, hh_ref, zh_ref):
    h = jnp.dot(x_ref[...], w1_ref[...], preferred_element_type=jnp.float32)
    h = h + b1_ref[...]
    inv = g_ref[...] * lax.rsqrt(v_ref[...] + 1e-5)
    h = (h - m_ref[...]) * inv + bb_ref[...]
    h = jnp.maximum(h, 0.0)
    o = jnp.dot(h, w2_ref[...], preferred_element_type=jnp.float32)
    o = o + b2_ref[...]
    hh_ref[0] = o[:, :HALF]
    hh_ref[1] = o[:, HALF:]
    zh_ref[0] = ALPHA * o[:, :HALF]
    zh_ref[1] = ALPHA * o[:, HALF:]


def _mlp(x, W1, b1, gamma, beta, mean, var, W2, b2):
    grid = (N // ROW_BLOCK,)
    full = lambda i: (0, 0)
    out_t = jax.ShapeDtypeStruct((NUM_CORES, N, HALF), jnp.float32)
    return pl.pallas_call(
        _mlp_body,
        grid=grid,
        in_specs=[
            pl.BlockSpec((ROW_BLOCK, D_IN), lambda i: (i, 0)),
            pl.BlockSpec((D_IN, D_HID), full),
            pl.BlockSpec((1, D_HID), full),
            pl.BlockSpec((1, D_HID), full),
            pl.BlockSpec((1, D_HID), full),
            pl.BlockSpec((1, D_HID), full),
            pl.BlockSpec((1, D_HID), full),
            pl.BlockSpec((D_HID, D_OUT), full),
            pl.BlockSpec((1, D_OUT), full),
        ],
        out_specs=[
            pl.BlockSpec((NUM_CORES, ROW_BLOCK, HALF), lambda i: (0, i, 0)),
            pl.BlockSpec((NUM_CORES, ROW_BLOCK, HALF), lambda i: (0, i, 0)),
        ],
        out_shape=[out_t, out_t],
    )(x, W1, b1.reshape(1, -1), gamma.reshape(1, -1), beta.reshape(1, -1),
      mean.reshape(1, -1), var.reshape(1, -1), W2, b2.reshape(1, -1))


def _logsoftmax_body(hf_ref, o_ref):
    h = jnp.concatenate([hf_ref[0], hf_ref[1]], axis=-1)
    m = jnp.max(h, axis=-1, keepdims=True)
    e = jnp.exp(h - m)
    lse = jnp.log(jnp.sum(e, axis=-1, keepdims=True)) + m
    o_ref[...] = h - lse


def _logsoftmax(hf):
    return pl.pallas_call(
        _logsoftmax_body,
        grid=(N // ROW_BLOCK,),
        in_specs=[pl.BlockSpec((NUM_CORES, ROW_BLOCK, HALF),
                               lambda i: (0, i, 0))],
        out_specs=pl.BlockSpec((ROW_BLOCK, D_OUT), lambda i: (i, 0)),
        out_shape=jax.ShapeDtypeStruct((N, D_OUT), jnp.float32),
    )(hf)


def _sc_compiler_params():
    cp = pltpu.CompilerParams()
    fields = pltpu.CompilerParams.__dataclass_fields__
    if "needs_layout_passes" in fields:
        cp = dataclasses.replace(cp, needs_layout_passes=False)
    if "use_tc_tiling_on_sc" in fields:
        cp = dataclasses.replace(cp, use_tc_tiling_on_sc=False)
    return cp


def _propagate(hh, zh, srcs, dsts, ws):
    mesh = plsc.VectorSubcoreMesh(core_axis_name="c", subcore_axis_name="s")

    @functools.partial(
        pl.kernel,
        out_type=jax.ShapeDtypeStruct((NUM_CORES, N_PAD, HALF), jnp.float32),
        mesh=mesh,
        compiler_params=_sc_compiler_params(),
        scratch_types=[
            pltpu.VMEM((CHUNK,), jnp.int32),
            pltpu.VMEM((CHUNK,), jnp.int32),
            pltpu.VMEM((CHUNK,), jnp.float32),
            pltpu.VMEM((CHUNK, HALF), jnp.float32),
            pltpu.VMEM_SHARED((N_PAD, HALF), jnp.float32),
            pltpu.SemaphoreType.DMA,
        ],
    )
    def prop(h0_hbm, z_hbm, src_hbm, dst_hbm, w_hbm, out_hbm,
             src_v, dst_v, w_v, rows_v, acc, sem):
        c = lax.axis_index("c")
        s = lax.axis_index("s")
        row0 = s * ROWS_PER_TILE
        chunk0 = s * CHUNKS_PER_TILE
        rows_slice = pl.ds(row0, ROWS_PER_TILE)

        for k in range(K):
            table = h0_hbm.at[c] if k == 0 else out_hbm.at[c]
            pltpu.sync_copy(z_hbm.at[c].at[rows_slice], acc.at[rows_slice])
            plsc.subcore_barrier()

            @pl.loop(0, CHUNKS_PER_TILE)
            def _(t):
                e0 = (chunk0 + t) * CHUNK
                pltpu.sync_copy(src_hbm.at[pl.ds(e0, CHUNK)], src_v)
                pltpu.sync_copy(dst_hbm.at[pl.ds(e0, CHUNK)], dst_v)
                pltpu.sync_copy(w_hbm.at[pl.ds(e0, CHUNK)], w_v)
                pltpu.async_copy(table.at[src_v], rows_v, sem).wait()

                @pl.loop(0, CHUNK)
                def _(j):
                    wb = plsc.load_gather(
                        w_v, [jnp.full((L,), j, jnp.int32)]) * (1.0 - ALPHA)
                    rows_v[j, pl.ds(0, L)] = rows_v[j, pl.ds(0, L)] * wb
                    rows_v[j, pl.ds(L, L)] = rows_v[j, pl.ds(L, L)] * wb

                pltpu.sync_copy(rows_v, acc.at[dst_v], add=True)

            plsc.subcore_barrier()
            pltpu.sync_copy(acc.at[rows_slice], out_hbm.at[c].at[rows_slice])
            plsc.subcore_barrier()

    return prop(hh, zh, srcs, dsts, ws)


def kernel(x, edge_index, edge_weight, W1, b1, gamma, beta, running_mean,
           running_var, W2, b2):
    dst = edge_index[0]
    src = edge_index[1]
    pad = E_PAD - E
    srcs = jnp.pad(src, (0, pad))
    dsts = jnp.pad(dst, (0, pad))
    ws = jnp.pad(edge_weight, (0, pad))

    hh, zh = _mlp(x, W1, b1, gamma, beta, running_mean, running_var, W2, b2)
    hh = jnp.pad(hh, ((0, 0), (0, N_PAD - N), (0, 0)))
    zh = jnp.pad(zh, ((0, 0), (0, N_PAD - N), (0, 0)))
    hf = _propagate(hh, zh, srcs, dsts, ws)
    return _logsoftmax(hf)
